# baseline (speedup 1.0000x reference)
import jax
import jax.numpy as jnp
from jax.experimental import pallas as pl

_BQ = 512


def _body(q_ref, k_ref, idx_ref, dist_ref):
    q = q_ref[...]
    k = k_ref[...]
    dots = jax.lax.dot_general(
        q, k, (((1,), (1,)), ((), ())), preferred_element_type=jnp.float32
    )
    q_sq = jnp.sum(q * q, axis=1, keepdims=True)
    k_sq = jnp.sum(k * k, axis=1)[None, :]
    s = jnp.maximum(q_sq + k_sq - 2.0 * dots, 1e-12)
    m = jnp.min(s, axis=1, keepdims=True)
    n_keys = s.shape[1]
    iota = jax.lax.broadcasted_iota(jnp.int32, s.shape, 1)
    idx = jnp.min(jnp.where(s == m, iota, n_keys), axis=1)
    idx_ref[0, 0, :] = idx
    dist_ref[0, 0, :] = jnp.sqrt(m[:, 0])


def kernel(queries, keys):
    Q, D = queries.shape
    K, _ = keys.shape
    grid = Q // _BQ
    idx, dist = pl.pallas_call(
        _body,
        grid=(grid,),
        in_specs=[
            pl.BlockSpec((_BQ, D), lambda i: (i, 0)),
            pl.BlockSpec((K, D), lambda i: (0, 0)),
        ],
        out_specs=[
            pl.BlockSpec((1, 1, _BQ), lambda i: (i, 0, 0)),
            pl.BlockSpec((1, 1, _BQ), lambda i: (i, 0, 0)),
        ],
        out_shape=[
            jax.ShapeDtypeStruct((grid, 1, _BQ), jnp.int32),
            jax.ShapeDtypeStruct((grid, 1, _BQ), jnp.float32),
        ],
    )(queries, keys)
    return idx.reshape(Q), dist.reshape(Q)
